# baseline (speedup 1.0000x reference)
import functools

import jax
import jax.numpy as jnp
from jax import lax
from jax.experimental import pallas as pl
from jax.experimental.pallas import tpu as pltpu
from jax.experimental.pallas import tpu_sc as plsc

N = 10000
NP = 10240
D_IN = 128
DH = 32
E = 320000
CHUNK = 3200
E_HALF = E // 2
NCHUNK = E_HALF // CHUNK
D_SLICE = E_HALF // 16
NZV = NP // 16
BLK = 2048
RBLK = 2048

_mesh = plsc.VectorSubcoreMesh(core_axis_name="c", subcore_axis_name="s")


@functools.partial(
    pl.kernel,
    out_type=(
        jax.ShapeDtypeStruct((2, 16, NP), jnp.float32),
        jax.ShapeDtypeStruct((2, DH, NP), jnp.float32),
    ),
    mesh=_mesh,
    compiler_params=pltpu.CompilerParams(needs_layout_passes=False),
    scratch_types=[
        pltpu.VMEM((NP,), jnp.float32),
        pltpu.VMEM((NP,), jnp.float32),
        pltpu.VMEM((NP,), jnp.float32),
        pltpu.VMEM((NP,), jnp.float32),
        pltpu.VMEM((NP,), jnp.float32),
        pltpu.VMEM((NP,), jnp.float32),
        pltpu.VMEM((NP,), jnp.float32),
        pltpu.VMEM((2 * CHUNK,), jnp.int32),
        pltpu.VMEM((2 * CHUNK,), jnp.int32),
        pltpu.VMEM((D_SLICE,), jnp.int32),
        pltpu.VMEM((D_SLICE,), jnp.int32),
        pltpu.SemaphoreType.DMA,
        pltpu.SemaphoreType.DMA,
        pltpu.SemaphoreType.DMA,
    ],
)
def _gat_aggregate(ei_hbm, a_s_hbm, a_d_hbm, hT_hbm,
                   denom_out, acc_out,
                   tabAS, tabAD, tabH0, tabH1, acc0, acc1, denom,
                   srcb, dstb, dsrcb, ddstb, sem0, sem1, sem2):
    c = lax.axis_index("c")
    s = lax.axis_index("s")
    L = hT_hbm.shape[1]

    cp0 = pltpu.async_copy(a_s_hbm.at[0], tabAS.at[pl.ds(0, L)], sem0)
    cp1 = pltpu.async_copy(a_d_hbm.at[0], tabAD.at[pl.ds(0, L)], sem0)
    cp2 = pltpu.async_copy(hT_hbm.at[s], tabH0.at[pl.ds(0, L)], sem0)
    cp3 = pltpu.async_copy(hT_hbm.at[s + 16], tabH1.at[pl.ds(0, L)], sem0)
    based = c * E_HALF + s * D_SLICE
    cp4 = pltpu.async_copy(ei_hbm.at[pl.ds(based, D_SLICE)], dsrcb, sem2)
    cp5 = pltpu.async_copy(ei_hbm.at[pl.ds(E + based, D_SLICE)], ddstb, sem2)

    zeros = jnp.zeros((16,), jnp.float32)

    @plsc.parallel_loop(0, NZV, unroll=4)
    def _zden(i):
        denom[pl.ds(i * 16, 16)] = zeros

    @pl.when(c == 1)
    def _zero():
        @plsc.parallel_loop(0, NZV, unroll=4)
        def _zbody(i):
            sl = pl.ds(i * 16, 16)
            acc0[sl] = zeros
            acc1[sl] = zeros

    cp0.wait()
    cp1.wait()
    cp2.wait()
    cp3.wait()

    @pl.when(c == 0)
    def _selfpass():
        @plsc.parallel_loop(0, N // 16, unroll=4)
        def _sbody(i):
            sl = pl.ds(i * 16, 16)
            e = tabAS[sl] + tabAD[sl]
            e = jnp.where(e >= 0.0, e, e * 0.2)
            w = jnp.exp(e)
            acc0[sl] = w * tabH0[sl]
            acc1[sl] = w * tabH1[sl]

    @pl.when(jnp.logical_and(c == 0, s == 0))
    def _selfdenom():
        @plsc.parallel_loop(0, N // 16, unroll=4)
        def _sdbody(i):
            sl = pl.ds(i * 16, 16)
            e = tabAS[sl] + tabAD[sl]
            e = jnp.where(e >= 0.0, e, e * 0.2)
            denom[sl] = jnp.exp(e)

    base = c * E_HALF
    sems = (sem0, sem1)

    def _issue(g, b):
        off = base + g * CHUNK
        slot = pl.ds(b * CHUNK, CHUNK)
        pltpu.async_copy(ei_hbm.at[pl.ds(off, CHUNK)], srcb.at[slot], sems[b])
        pltpu.async_copy(ei_hbm.at[pl.ds(E + off, CHUNK)], dstb.at[slot], sems[b])

    def _wait(g, b):
        off = base + g * CHUNK
        slot = pl.ds(b * CHUNK, CHUNK)
        pltpu.make_async_copy(ei_hbm.at[pl.ds(off, CHUNK)], srcb.at[slot], sems[b]).wait()
        pltpu.make_async_copy(ei_hbm.at[pl.ds(E + off, CHUNK)], dstb.at[slot], sems[b]).wait()

    _issue(0, 0)
    _issue(1, 1)

    cp4.wait()
    cp5.wait()

    @plsc.parallel_loop(0, D_SLICE // 16, unroll=4)
    def _dbody(i):
        sl = pl.ds(i * 16, 16)
        sv = dsrcb[sl]
        dv = ddstb[sl]
        e = plsc.load_gather(tabAS, [sv]) + plsc.load_gather(tabAD, [dv])
        e = jnp.where(e >= 0.0, e, e * 0.2)
        plsc.addupdate_scatter(denom, [dv], jnp.exp(e))

    pltpu.sync_copy(denom, denom_out.at[c, s])

    def _compute(b):
        sbase = b * CHUNK

        @plsc.parallel_loop(0, CHUNK // 16, unroll=8)
        def _body(i):
            sl = pl.ds(sbase + i * 16, 16)
            sv = srcb[sl]
            dv = dstb[sl]
            ga = plsc.load_gather(tabAS, [sv])
            gd = plsc.load_gather(tabAD, [dv])
            h0 = plsc.load_gather(tabH0, [sv])
            h1 = plsc.load_gather(tabH1, [sv])
            e = ga + gd
            e = jnp.where(e >= 0.0, e, e * 0.2)
            w = jnp.exp(e)
            plsc.addupdate_scatter(acc0, [dv], h0 * w)
            plsc.addupdate_scatter(acc1, [dv], h1 * w)

    def mbody(g2, carry):
        g = g2 * 2
        _wait(g, 0)
        _compute(0)
        _issue(g + 2, 0)
        _wait(g + 1, 1)
        _compute(1)
        _issue(g + 3, 1)
        return carry

    lax.fori_loop(0, NCHUNK // 2 - 1, mbody, 0)

    _wait(NCHUNK - 2, 0)
    _compute(0)
    _wait(NCHUNK - 1, 1)
    _compute(1)

    pltpu.sync_copy(acc0, acc_out.at[c, s])
    pltpu.sync_copy(acc1, acc_out.at[c, s + 16])


def _proj_body(x_ref, W_ref, att_s_ref, att_d_ref, hT_ref, as_ref, ad_ref):
    hT = lax.dot_general(W_ref[...], x_ref[...], (((0,), (1,)), ((), ())),
                         preferred_element_type=jnp.float32)
    hT_ref[...] = hT
    as_ref[...] = jnp.dot(att_s_ref[...], hT, preferred_element_type=jnp.float32)
    ad_ref[...] = jnp.dot(att_d_ref[...], hT, preferred_element_type=jnp.float32)


def _proj(x, W, att_s, att_d):
    return pl.pallas_call(
        _proj_body,
        grid=(pl.cdiv(N, RBLK),),
        in_specs=[
            pl.BlockSpec((RBLK, D_IN), lambda j: (j, 0)),
            pl.BlockSpec((D_IN, DH), lambda j: (0, 0)),
            pl.BlockSpec((1, DH), lambda j: (0, 0)),
            pl.BlockSpec((1, DH), lambda j: (0, 0)),
        ],
        out_specs=[
            pl.BlockSpec((DH, RBLK), lambda j: (0, j)),
            pl.BlockSpec((1, RBLK), lambda j: (0, j)),
            pl.BlockSpec((1, RBLK), lambda j: (0, j)),
        ],
        out_shape=[
            jax.ShapeDtypeStruct((DH, NP), jnp.float32),
            jax.ShapeDtypeStruct((1, NP), jnp.float32),
            jax.ShapeDtypeStruct((1, NP), jnp.float32),
        ],
    )(x, W, att_s, att_d)


def _norm_proj_body(acc_ref, den_ref, b_ref, WT_ref, att_s_ref, att_d_ref,
                    hT_ref, as_ref, ad_ref):
    accs = acc_ref[0] + acc_ref[1]
    den = jnp.sum(den_ref[0] + den_ref[1], axis=0, keepdims=True)
    h = jnp.maximum(accs / (den + 1e-16) + b_ref[...], 0.0)
    hT2 = jnp.dot(WT_ref[...], h, preferred_element_type=jnp.float32)
    hT_ref[...] = hT2
    as_ref[...] = jnp.dot(att_s_ref[...], hT2, preferred_element_type=jnp.float32)
    ad_ref[...] = jnp.dot(att_d_ref[...], hT2, preferred_element_type=jnp.float32)


def _norm_proj(acc, den, b, WT, att_s, att_d):
    return pl.pallas_call(
        _norm_proj_body,
        grid=(NP // BLK,),
        in_specs=[
            pl.BlockSpec((2, DH, BLK), lambda j: (0, 0, j)),
            pl.BlockSpec((2, 16, BLK), lambda j: (0, 0, j)),
            pl.BlockSpec((DH, 1), lambda j: (0, 0)),
            pl.BlockSpec((DH, DH), lambda j: (0, 0)),
            pl.BlockSpec((1, DH), lambda j: (0, 0)),
            pl.BlockSpec((1, DH), lambda j: (0, 0)),
        ],
        out_specs=[
            pl.BlockSpec((DH, BLK), lambda j: (0, j)),
            pl.BlockSpec((1, BLK), lambda j: (0, j)),
            pl.BlockSpec((1, BLK), lambda j: (0, j)),
        ],
        out_shape=[
            jax.ShapeDtypeStruct((DH, NP), jnp.float32),
            jax.ShapeDtypeStruct((1, NP), jnp.float32),
            jax.ShapeDtypeStruct((1, NP), jnp.float32),
        ],
    )(acc, den, b, WT, att_s, att_d)


def _final_body(acc_ref, den_ref, b_ref, Wf_ref, bf_ref, out_ref):
    accs = acc_ref[0] + acc_ref[1]
    den = jnp.sum(den_ref[0] + den_ref[1], axis=0, keepdims=True)
    h = jnp.maximum(accs / (den + 1e-16) + b_ref[...], 0.0)
    out_ref[...] = lax.dot_general(
        h, Wf_ref[...], (((0,), (0,)), ((), ())),
        preferred_element_type=jnp.float32) + bf_ref[...]


def _final(acc, den, b, Wf, bf):
    return pl.pallas_call(
        _final_body,
        grid=(pl.cdiv(N, RBLK),),
        in_specs=[
            pl.BlockSpec((2, DH, RBLK), lambda j: (0, 0, j)),
            pl.BlockSpec((2, 16, RBLK), lambda j: (0, 0, j)),
            pl.BlockSpec((DH, 1), lambda j: (0, 0)),
            pl.BlockSpec((DH, DH), lambda j: (0, 0)),
            pl.BlockSpec((1, DH), lambda j: (0, 0)),
        ],
        out_specs=pl.BlockSpec((RBLK, DH), lambda j: (j, 0)),
        out_shape=jax.ShapeDtypeStruct((N, DH), jnp.float32),
    )(acc, den, b, Wf, bf)


def kernel(x, edge_index, W1, att_src1, att_dst1, b1,
           W2, att_src2, att_dst2, b2, Wf, bf):
    hT1, as1, ad1 = _proj(x, W1, att_src1.reshape(1, DH), att_dst1.reshape(1, DH))
    ei = edge_index.reshape(2 * E)
    den1, acc1 = _gat_aggregate(ei, as1, ad1, hT1)

    hT2, as2, ad2 = _norm_proj(acc1, den1, b1.reshape(DH, 1), W2.T,
                               att_src2.reshape(1, DH), att_dst2.reshape(1, DH))
    den2, acc2 = _gat_aggregate(ei, as2, ad2, hT2)

    return _final(acc2, den2, b2.reshape(DH, 1), Wf, bf.reshape(1, DH))


# speedup vs baseline: 1.1003x; 1.1003x over previous
import functools

import jax
import jax.numpy as jnp
from jax import lax
from jax.experimental import pallas as pl
from jax.experimental.pallas import tpu as pltpu
from jax.experimental.pallas import tpu_sc as plsc

N = 10000
NP = 10240
D_IN = 128
DH = 32
E = 320000
CHUNK = 3200
E_HALF = E // 2
NCHUNK = E_HALF // CHUNK
D_SLICE = E_HALF // 16
NZV = NP // 16
BLK = 2048
RBLK = 2048

_mesh = plsc.VectorSubcoreMesh(core_axis_name="c", subcore_axis_name="s")


@functools.partial(
    pl.kernel,
    out_type=(
        jax.ShapeDtypeStruct((2, 16, NP), jnp.float32),
        jax.ShapeDtypeStruct((2, DH, NP), jnp.float32),
    ),
    mesh=_mesh,
    compiler_params=pltpu.CompilerParams(needs_layout_passes=False),
    scratch_types=[
        pltpu.VMEM((NP,), jnp.float32),
        pltpu.VMEM((NP,), jnp.float32),
        pltpu.VMEM((NP,), jnp.int32),
        pltpu.VMEM((NP,), jnp.float32),
        pltpu.VMEM((NP,), jnp.float32),
        pltpu.VMEM((NP,), jnp.float32),
        pltpu.VMEM((2 * CHUNK,), jnp.int32),
        pltpu.VMEM((2 * CHUNK,), jnp.int32),
        pltpu.VMEM((D_SLICE,), jnp.int32),
        pltpu.VMEM((D_SLICE,), jnp.int32),
        pltpu.SemaphoreType.DMA,
        pltpu.SemaphoreType.DMA,
        pltpu.SemaphoreType.DMA,
    ],
)
def _gat_aggregate(ei_hbm, a_s_hbm, a_d_hbm, hP_hbm,
                   denom_out, acc_out,
                   tabAS, tabAD, tabHP, acc0, acc1, denom,
                   srcb, dstb, dsrcb, ddstb, sem0, sem1, sem2):
    c = lax.axis_index("c")
    s = lax.axis_index("s")
    L = hP_hbm.shape[1]

    cp0 = pltpu.async_copy(a_s_hbm.at[0], tabAS.at[pl.ds(0, L)], sem0)
    cp1 = pltpu.async_copy(a_d_hbm.at[0], tabAD.at[pl.ds(0, L)], sem0)
    cp2 = pltpu.async_copy(hP_hbm.at[s], tabHP.at[pl.ds(0, L)], sem0)
    based = c * E_HALF + s * D_SLICE
    cp4 = pltpu.async_copy(ei_hbm.at[pl.ds(based, D_SLICE)], dsrcb, sem2)
    cp5 = pltpu.async_copy(ei_hbm.at[pl.ds(E + based, D_SLICE)], ddstb, sem2)

    zeros = jnp.zeros((16,), jnp.float32)

    @plsc.parallel_loop(0, NZV, unroll=4)
    def _zden(i):
        denom[pl.ds(i * 16, 16)] = zeros

    @pl.when(c == 1)
    def _zero():
        @plsc.parallel_loop(0, NZV, unroll=4)
        def _zbody(i):
            sl = pl.ds(i * 16, 16)
            acc0[sl] = zeros
            acc1[sl] = zeros

    cp0.wait()
    cp1.wait()
    cp2.wait()

    def _unpack(gp):
        h0 = plsc.bitcast(lax.shift_left(gp, 16), jnp.float32)
        h1 = plsc.bitcast(lax.bitwise_and(gp, jnp.int32(-65536)), jnp.float32)
        return h0, h1

    @pl.when(c == 0)
    def _selfpass():
        @plsc.parallel_loop(0, N // 16, unroll=4)
        def _sbody(i):
            sl = pl.ds(i * 16, 16)
            e = tabAS[sl] + tabAD[sl]
            e = jnp.where(e >= 0.0, e, e * 0.2)
            w = jnp.exp(e)
            h0, h1 = _unpack(tabHP[sl])
            acc0[sl] = w * h0
            acc1[sl] = w * h1

    @pl.when(jnp.logical_and(c == 0, s == 0))
    def _selfdenom():
        @plsc.parallel_loop(0, N // 16, unroll=4)
        def _sdbody(i):
            sl = pl.ds(i * 16, 16)
            e = tabAS[sl] + tabAD[sl]
            e = jnp.where(e >= 0.0, e, e * 0.2)
            denom[sl] = jnp.exp(e)

    base = c * E_HALF
    sems = (sem0, sem1)

    def _issue(g, b):
        off = base + g * CHUNK
        slot = pl.ds(b * CHUNK, CHUNK)
        pltpu.async_copy(ei_hbm.at[pl.ds(off, CHUNK)], srcb.at[slot], sems[b])
        pltpu.async_copy(ei_hbm.at[pl.ds(E + off, CHUNK)], dstb.at[slot], sems[b])

    def _wait(g, b):
        off = base + g * CHUNK
        slot = pl.ds(b * CHUNK, CHUNK)
        pltpu.make_async_copy(ei_hbm.at[pl.ds(off, CHUNK)], srcb.at[slot], sems[b]).wait()
        pltpu.make_async_copy(ei_hbm.at[pl.ds(E + off, CHUNK)], dstb.at[slot], sems[b]).wait()

    _issue(0, 0)
    _issue(1, 1)

    cp4.wait()
    cp5.wait()

    @plsc.parallel_loop(0, D_SLICE // 16, unroll=4)
    def _dbody(i):
        sl = pl.ds(i * 16, 16)
        sv = dsrcb[sl]
        dv = ddstb[sl]
        e = plsc.load_gather(tabAS, [sv]) + plsc.load_gather(tabAD, [dv])
        e = jnp.where(e >= 0.0, e, e * 0.2)
        plsc.addupdate_scatter(denom, [dv], jnp.exp(e))

    pltpu.sync_copy(denom, denom_out.at[c, s])

    def _compute(b):
        sbase = b * CHUNK

        @plsc.parallel_loop(0, CHUNK // 16, unroll=4)
        def _body(i):
            sl = pl.ds(sbase + i * 16, 16)
            sv = srcb[sl]
            dv = dstb[sl]
            ga = plsc.load_gather(tabAS, [sv])
            gd = plsc.load_gather(tabAD, [dv])
            h0, h1 = _unpack(plsc.load_gather(tabHP, [sv]))
            e = ga + gd
            e = jnp.where(e >= 0.0, e, e * 0.2)
            w = jnp.exp(e)
            plsc.addupdate_scatter(acc0, [dv], h0 * w)
            plsc.addupdate_scatter(acc1, [dv], h1 * w)

    def mbody(g2, carry):
        g = g2 * 2
        _wait(g, 0)
        _compute(0)
        _issue(g + 2, 0)
        _wait(g + 1, 1)
        _compute(1)
        _issue(g + 3, 1)
        return carry

    lax.fori_loop(0, NCHUNK // 2 - 1, mbody, 0)

    _wait(NCHUNK - 2, 0)
    _compute(0)
    _wait(NCHUNK - 1, 1)
    _compute(1)

    pltpu.sync_copy(acc0, acc_out.at[c, s])
    pltpu.sync_copy(acc1, acc_out.at[c, s + 16])


def _pack_pair(hT):
    lo = lax.convert_element_type(hT[:16, :], jnp.bfloat16)
    hi = lax.convert_element_type(hT[16:, :], jnp.bfloat16)
    lo32 = lax.convert_element_type(
        lax.bitcast_convert_type(lo, jnp.uint16), jnp.uint32)
    hi32 = lax.convert_element_type(
        lax.bitcast_convert_type(hi, jnp.uint16), jnp.uint32)
    return lax.bitcast_convert_type(
        lax.bitwise_or(lo32, lax.shift_left(hi32, jnp.uint32(16))), jnp.int32)


def _proj_body(x_ref, W_ref, att_s_ref, att_d_ref, hP_ref, as_ref, ad_ref):
    hT = lax.dot_general(W_ref[...], x_ref[...], (((0,), (1,)), ((), ())),
                         preferred_element_type=jnp.float32)
    hP_ref[...] = _pack_pair(hT)
    as_ref[...] = jnp.dot(att_s_ref[...], hT, preferred_element_type=jnp.float32)
    ad_ref[...] = jnp.dot(att_d_ref[...], hT, preferred_element_type=jnp.float32)


def _proj(x, W, att_s, att_d):
    return pl.pallas_call(
        _proj_body,
        grid=(pl.cdiv(N, RBLK),),
        in_specs=[
            pl.BlockSpec((RBLK, D_IN), lambda j: (j, 0)),
            pl.BlockSpec((D_IN, DH), lambda j: (0, 0)),
            pl.BlockSpec((1, DH), lambda j: (0, 0)),
            pl.BlockSpec((1, DH), lambda j: (0, 0)),
        ],
        out_specs=[
            pl.BlockSpec((16, RBLK), lambda j: (0, j)),
            pl.BlockSpec((1, RBLK), lambda j: (0, j)),
            pl.BlockSpec((1, RBLK), lambda j: (0, j)),
        ],
        out_shape=[
            jax.ShapeDtypeStruct((16, NP), jnp.int32),
            jax.ShapeDtypeStruct((1, NP), jnp.float32),
            jax.ShapeDtypeStruct((1, NP), jnp.float32),
        ],
    )(x, W, att_s, att_d)


def _norm_proj_body(acc_ref, den_ref, b_ref, WT_ref, att_s_ref, att_d_ref,
                    hP_ref, as_ref, ad_ref):
    accs = acc_ref[0] + acc_ref[1]
    den = jnp.sum(den_ref[0] + den_ref[1], axis=0, keepdims=True)
    h = jnp.maximum(accs / (den + 1e-16) + b_ref[...], 0.0)
    hT2 = jnp.dot(WT_ref[...], h, preferred_element_type=jnp.float32)
    hP_ref[...] = _pack_pair(hT2)
    as_ref[...] = jnp.dot(att_s_ref[...], hT2, preferred_element_type=jnp.float32)
    ad_ref[...] = jnp.dot(att_d_ref[...], hT2, preferred_element_type=jnp.float32)


def _norm_proj(acc, den, b, WT, att_s, att_d):
    return pl.pallas_call(
        _norm_proj_body,
        grid=(NP // BLK,),
        in_specs=[
            pl.BlockSpec((2, DH, BLK), lambda j: (0, 0, j)),
            pl.BlockSpec((2, 16, BLK), lambda j: (0, 0, j)),
            pl.BlockSpec((DH, 1), lambda j: (0, 0)),
            pl.BlockSpec((DH, DH), lambda j: (0, 0)),
            pl.BlockSpec((1, DH), lambda j: (0, 0)),
            pl.BlockSpec((1, DH), lambda j: (0, 0)),
        ],
        out_specs=[
            pl.BlockSpec((16, BLK), lambda j: (0, j)),
            pl.BlockSpec((1, BLK), lambda j: (0, j)),
            pl.BlockSpec((1, BLK), lambda j: (0, j)),
        ],
        out_shape=[
            jax.ShapeDtypeStruct((16, NP), jnp.int32),
            jax.ShapeDtypeStruct((1, NP), jnp.float32),
            jax.ShapeDtypeStruct((1, NP), jnp.float32),
        ],
    )(acc, den, b, WT, att_s, att_d)


def _final_body(acc_ref, den_ref, b_ref, Wf_ref, bf_ref, out_ref):
    accs = acc_ref[0] + acc_ref[1]
    den = jnp.sum(den_ref[0] + den_ref[1], axis=0, keepdims=True)
    h = jnp.maximum(accs / (den + 1e-16) + b_ref[...], 0.0)
    out_ref[...] = lax.dot_general(
        h, Wf_ref[...], (((0,), (0,)), ((), ())),
        preferred_element_type=jnp.float32) + bf_ref[...]


def _final(acc, den, b, Wf, bf):
    return pl.pallas_call(
        _final_body,
        grid=(pl.cdiv(N, RBLK),),
        in_specs=[
            pl.BlockSpec((2, DH, RBLK), lambda j: (0, 0, j)),
            pl.BlockSpec((2, 16, RBLK), lambda j: (0, 0, j)),
            pl.BlockSpec((DH, 1), lambda j: (0, 0)),
            pl.BlockSpec((DH, DH), lambda j: (0, 0)),
            pl.BlockSpec((1, DH), lambda j: (0, 0)),
        ],
        out_specs=pl.BlockSpec((RBLK, DH), lambda j: (j, 0)),
        out_shape=jax.ShapeDtypeStruct((N, DH), jnp.float32),
    )(acc, den, b, Wf, bf)


def kernel(x, edge_index, W1, att_src1, att_dst1, b1,
           W2, att_src2, att_dst2, b2, Wf, bf):
    hT1, as1, ad1 = _proj(x, W1, att_src1.reshape(1, DH), att_dst1.reshape(1, DH))
    ei = edge_index.reshape(2 * E)
    den1, acc1 = _gat_aggregate(ei, as1, ad1, hT1)

    hT2, as2, ad2 = _norm_proj(acc1, den1, b1.reshape(DH, 1), W2.T,
                               att_src2.reshape(1, DH), att_dst2.reshape(1, DH))
    den2, acc2 = _gat_aggregate(ei, as2, ad2, hT2)

    return _final(acc2, den2, b2.reshape(DH, 1), Wf, bf.reshape(1, DH))
